# Initial kernel scaffold; baseline (speedup 1.0000x reference)
import jax
import jax.numpy as jnp
from jax.experimental import pallas as pl


def kernel(x, edge_index, edge_weight, W1, b1, W2, b2, W3, b3, A1, ab1, A2, ab2, A3, ab3, A4, ab4):
    raise NotImplementedError("write your pallas kernel here")


# speedup vs baseline: 1.0252x; 1.0252x over previous
import functools

import jax
import jax.numpy as jnp
from jax.experimental import pallas as pl
from jax.experimental.pallas import tpu as pltpu

N = 50000
E = 800000
DIN = 12
H = 128
DOUT = 2

ROW_BLK = 2048


def _dense1_body(xa_ref, W1_ref, b1_ref, W2_ref, o_ref):
    h1 = jnp.maximum(
        jnp.dot(xa_ref[...], W1_ref[...], preferred_element_type=jnp.float32, precision=jax.lax.Precision.HIGHEST)
        + b1_ref[...], 0.0)
    o_ref[...] = jnp.dot(h1, W2_ref[...], preferred_element_type=jnp.float32, precision=jax.lax.Precision.HIGHEST)


def _dense2_body(a_ref, b_ref, W_ref, o_ref):
    h = jnp.maximum(a_ref[...] + b_ref[...], 0.0)
    o_ref[...] = jnp.dot(h, W_ref[...], preferred_element_type=jnp.float32, precision=jax.lax.Precision.HIGHEST)


def _head_body(a3_ref, b3_ref, A1_ref, ab1_ref, A2_ref, ab2_ref, A3_ref,
               ab3_ref, A4_ref, ab4_ref, o_ref):
    h = jnp.maximum(a3_ref[...] + b3_ref[...], 0.0)
    h = jnp.maximum(
        jnp.dot(h, A1_ref[...], preferred_element_type=jnp.float32, precision=jax.lax.Precision.HIGHEST)
        + ab1_ref[...], 0.0)
    h = jnp.maximum(
        jnp.dot(h, A2_ref[...], preferred_element_type=jnp.float32, precision=jax.lax.Precision.HIGHEST)
        + ab2_ref[...], 0.0)
    h = jnp.maximum(
        jnp.dot(h, A3_ref[...], preferred_element_type=jnp.float32, precision=jax.lax.Precision.HIGHEST)
        + ab3_ref[...], 0.0)
    o_ref[...] = (jnp.dot(h, A4_ref[...], preferred_element_type=jnp.float32, precision=jax.lax.Precision.HIGHEST)
                  + ab4_ref[...])


def _row_blocked(body, n_rows, out_dim, n_in_blocked, x, *full_args):
    grid = (pl.cdiv(n_rows, ROW_BLK),)
    in_specs = [pl.BlockSpec((ROW_BLK, x.shape[1]), lambda i: (i, 0))]
    for a in full_args:
        in_specs.append(pl.BlockSpec(a.shape, lambda i: tuple(0 for _ in a.shape)))
    return pl.pallas_call(
        body,
        grid=grid,
        in_specs=in_specs,
        out_specs=pl.BlockSpec((ROW_BLK, out_dim), lambda i: (i, 0)),
        out_shape=jax.ShapeDtypeStruct((n_rows, out_dim), jnp.float32),
    )(x, *full_args)


def _spmm(support, src, dst, w):
    msg = w[:, None] * support[src]
    return jax.ops.segment_sum(msg, dst, num_segments=N)


def kernel(x, edge_index, edge_weight, W1, b1, W2, b2, W3, b3,
           A1, ab1, A2, ab2, A3, ab3, A4, ab4):
    src = edge_index[1]
    dst = edge_index[0]

    xa = _spmm(x, src, dst, edge_weight)
    s2 = _row_blocked(_dense1_body, N, H, 1, xa,
                      W1, b1.reshape(1, H), W2)
    a2 = _spmm(s2, src, dst, edge_weight)
    s3 = _row_blocked(_dense2_body, N, H, 1, a2,
                      b2.reshape(1, H), W3)
    a3 = _spmm(s3, src, dst, edge_weight)
    scores = _row_blocked(_head_body, N, DOUT, 1, a3,
                          b3.reshape(1, H), A1, ab1.reshape(1, H),
                          A2, ab2.reshape(1, H), A3, ab3.reshape(1, H),
                          A4, ab4.reshape(1, DOUT))
    return scores


# speedup vs baseline: 2.5323x; 2.4700x over previous
import dataclasses
import functools

import jax
import jax.numpy as jnp
from jax import lax
from jax.experimental import pallas as pl
from jax.experimental.pallas import tpu as pltpu
from jax.experimental.pallas import tpu_sc as plsc

N = 50000
E = 800000
DIN = 12
H = 128
DOUT = 2

N_PAD = 51200
E_PAD = 819200
EPT = 51200
SCAN = 3200
NCHUNK = EPT // SCAN
NVEC = SCAN // 16
BLK = 128

ROW_BLK = 2048


def _dense0_body(x_ref, W_ref, o_ref):
    o_ref[...] = jnp.dot(x_ref[...], W_ref[...],
                         preferred_element_type=jnp.float32,
                         precision=lax.Precision.HIGHEST)


def _dense2_body(a_ref, b_ref, W_ref, o_ref):
    h = jnp.maximum(a_ref[...] + b_ref[...], 0.0)
    o_ref[...] = jnp.dot(h, W_ref[...], preferred_element_type=jnp.float32,
                         precision=lax.Precision.HIGHEST)


def _head_body(a3_ref, b3_ref, A1_ref, ab1_ref, A2_ref, ab2_ref, A3_ref,
               ab3_ref, A4_ref, ab4_ref, o_ref):
    h = jnp.maximum(a3_ref[...] + b3_ref[...], 0.0)
    for W_ref, b_ref in ((A1_ref, ab1_ref), (A2_ref, ab2_ref),
                         (A3_ref, ab3_ref)):
        h = jnp.maximum(
            jnp.dot(h, W_ref[...], preferred_element_type=jnp.float32,
                    precision=lax.Precision.HIGHEST) + b_ref[...], 0.0)
    o_ref[...] = (jnp.dot(h, A4_ref[...], preferred_element_type=jnp.float32,
                          precision=lax.Precision.HIGHEST) + ab4_ref[...])


def _row_blocked(body, out_dim, x, *full_args):
    grid = (N_PAD // ROW_BLK,)
    in_specs = [pl.BlockSpec((ROW_BLK, x.shape[1]), lambda i: (i, 0))]
    for a in full_args:
        in_specs.append(
            pl.BlockSpec(a.shape, lambda i, _r=len(a.shape): (0,) * _r))
    return pl.pallas_call(
        body,
        grid=grid,
        in_specs=in_specs,
        out_specs=pl.BlockSpec((ROW_BLK, out_dim), lambda i: (i, 0)),
        out_shape=jax.ShapeDtypeStruct((N_PAD, out_dim), jnp.float32),
    )(x, *full_args)


def _make_spmm(D, RANGE, NPASS, CAP):
    ROWS_PT = RANGE // 16
    CALLOC = CAP + 144
    NQ = D // 16
    mesh = plsc.VectorSubcoreMesh(core_axis_name="c", subcore_axis_name="s",
                                  num_cores=2, num_subcores=16)

    def body(S_hbm, dst_hbm, src_hbm, w_hbm, z_hbm, out_hbm,
             dstbuf, srcbuf, wbuf, st_src, st_drel, st_w,
             fsrc, fidx, rowbuf, acc):
        c = lax.axis_index("c")
        s = lax.axis_index("s")
        ebase = s * EPT
        iota = lax.iota(jnp.int32, 16)
        zi = jnp.zeros((16,), jnp.int32)
        zf = jnp.zeros((16,), jnp.float32)
        for p in range(NPASS):
            lo = (c * NPASS + p) * RANGE
            pltpu.sync_copy(z_hbm.at[pl.ds(s * ROWS_PT, ROWS_PT)],
                            acc.at[pl.ds(s * ROWS_PT, ROWS_PT)])
            plsc.subcore_barrier()

            def vec_body(j, ptr, _lo=lo):
                b = j * 16
                d = dstbuf[pl.ds(b, 16)]
                sv = srcbuf[pl.ds(b, 16)]
                wv = wbuf[pl.ds(b, 16)]
                drel = d - _lo
                m = (drel >= 0) & (drel < RANGE)
                mi = jnp.where(m, 1, 0).astype(jnp.int32)
                inc = plsc.cumsum(mi)
                pos = ptr + inc - 1
                plsc.store_scatter(st_src, [pos], sv, mask=m)
                plsc.store_scatter(st_drel, [pos], drel, mask=m)
                plsc.store_scatter(st_w, [pos], wv, mask=m)
                cnt = jnp.sum(mi)
                return jnp.minimum(ptr + cnt, CAP)

            def chunk_body(ci, ptr):
                off = ebase + ci * SCAN
                pltpu.sync_copy(dst_hbm.at[pl.ds(off, SCAN)], dstbuf)
                pltpu.sync_copy(src_hbm.at[pl.ds(off, SCAN)], srcbuf)
                pltpu.sync_copy(w_hbm.at[pl.ds(off, SCAN)], wbuf)
                return lax.fori_loop(0, NVEC, vec_body, ptr)

            ptr = lax.fori_loop(0, NCHUNK, chunk_body, jnp.int32(0))

            p0 = (ptr // 16) * 16
            for q in range(8):
                idx16 = iota + p0 + q * 16
                mq = idx16 >= ptr
                plsc.store_scatter(st_src, [idx16], zi, mask=mq)
                plsc.store_scatter(st_drel, [idx16], zi, mask=mq)
                plsc.store_scatter(st_w, [idx16], zf, mask=mq)
            nblk = (ptr + BLK - 1) // BLK

            def blk_body(k, _):
                kb = k * BLK
                for q2 in range(BLK // 16):
                    fsrc[pl.ds(q2 * 16, 16)] = st_src[pl.ds(kb + q2 * 16, 16)]
                    fidx[pl.ds(q2 * 16, 16)] = st_drel[pl.ds(kb + q2 * 16, 16)]
                pltpu.sync_copy(S_hbm.at[fsrc], rowbuf)

                def row_body(i, _2, _kb=kb):
                    wrow = plsc.load_gather(
                        st_w, [jnp.full((16,), _kb + i, jnp.int32)])
                    for q3 in range(NQ):
                        rowbuf[i, pl.ds(q3 * 16, 16)] = (
                            rowbuf[i, pl.ds(q3 * 16, 16)] * wrow)
                    return 0

                lax.fori_loop(0, BLK, row_body, 0)
                pltpu.sync_copy(rowbuf, acc.at[fidx], add=True)
                return 0

            lax.fori_loop(0, nblk, blk_body, 0)
            plsc.subcore_barrier()

            pltpu.sync_copy(acc.at[pl.ds(s * ROWS_PT, ROWS_PT)],
                            out_hbm.at[pl.ds(lo + s * ROWS_PT, ROWS_PT)])
            plsc.subcore_barrier()

    cp = pltpu.CompilerParams()
    if "needs_layout_passes" in pltpu.CompilerParams.__dataclass_fields__:
        cp = dataclasses.replace(cp, needs_layout_passes=False)
    kern = pl.kernel(
        body,
        out_type=jax.ShapeDtypeStruct((N_PAD, D), jnp.float32),
        mesh=mesh,
        compiler_params=cp,
        scratch_types=[
            pltpu.VMEM((SCAN,), jnp.int32),
            pltpu.VMEM((SCAN,), jnp.int32),
            pltpu.VMEM((SCAN,), jnp.float32),
            pltpu.VMEM((CALLOC,), jnp.int32),
            pltpu.VMEM((CALLOC,), jnp.int32),
            pltpu.VMEM((CALLOC,), jnp.float32),
            pltpu.VMEM((BLK,), jnp.int32),
            pltpu.VMEM((BLK,), jnp.int32),
            pltpu.VMEM((BLK, D), jnp.float32),
            pltpu.VMEM_SHARED((RANGE, D), jnp.float32),
        ],
    )
    return kern


_spmm128 = _make_spmm(128, 6400, 4, 8192)


def kernel(x, edge_index, edge_weight, W1, b1, W2, b2, W3, b3,
           A1, ab1, A2, ab2, A3, ab3, A4, ab4):
    dst = jnp.concatenate([edge_index[0],
                           jnp.zeros((E_PAD - E,), jnp.int32)])
    src = jnp.concatenate([edge_index[1],
                           jnp.zeros((E_PAD - E,), jnp.int32)])
    w = jnp.concatenate([edge_weight, jnp.zeros((E_PAD - E,), jnp.float32)])
    x_pad = jnp.zeros((N_PAD, 16), jnp.float32).at[:N, :DIN].set(x)
    z128 = jnp.zeros((6400, 128), jnp.float32)
    W1p = jnp.zeros((16, H), jnp.float32).at[:DIN].set(W1)

    s1 = _row_blocked(_dense0_body, H, x_pad, W1p)
    a1 = _spmm128(s1, dst, src, w, z128)
    s2 = _row_blocked(_dense2_body, H, a1, b1.reshape(1, H), W2)
    a2 = _spmm128(s2, dst, src, w, z128)
    s3 = _row_blocked(_dense2_body, H, a2, b2.reshape(1, H), W3)
    a3 = _spmm128(s3, dst, src, w, z128)
    scores = _row_blocked(_head_body, DOUT, a3,
                          b3.reshape(1, H), A1, ab1.reshape(1, H),
                          A2, ab2.reshape(1, H), A3, ab3.reshape(1, H),
                          A4, ab4.reshape(1, DOUT))
    return scores[:N]
